# baseline (speedup 1.0000x reference)
import functools

import jax
import jax.numpy as jnp
from jax import lax
from jax.experimental import pallas as pl
from jax.experimental.pallas import tpu as pltpu
from jax.experimental.pallas import tpu_sc as plsc

N_NODES = 10000
IN_DIM = 128
HID_DIM = 256
OUT_DIM = 256

N_PAD = 10240
N_TILES = 32
OWN = N_PAD // N_TILES

STRIP = 3072
CAP = STRIP + 144
SUB = 48
E_PAD = 331776
E_PER_TILE = E_PAD // 32
N_STRIPS = E_PAD // STRIP
E_ALLOC = E_PAD + 2 * STRIP
JUNK_ROW = N_NODES + 16


def _mesh():
    return plsc.VectorSubcoreMesh(core_axis_name="c", subcore_axis_name="s")


@functools.partial(
    pl.kernel,
    mesh=_mesh(),
    out_type=jax.ShapeDtypeStruct((N_TILES, N_PAD), jnp.float32),
    scratch_types=[
        pltpu.VMEM((E_PER_TILE,), jnp.int32),
        pltpu.VMEM((N_PAD,), jnp.float32),
    ],
    compiler_params=pltpu.CompilerParams(needs_layout_passes=False),
)
def _deg_sc(dst_hbm, out_hbm, dst_all, hist_v):
    c = lax.axis_index("c")
    s = lax.axis_index("s")
    wid = c * 16 + s
    e0 = wid * E_PER_TILE
    pltpu.sync_copy(dst_hbm.at[pl.ds(e0, E_PER_TILE)], dst_all)

    zeros16 = jnp.zeros((16,), jnp.float32)
    ones16 = jnp.ones((16,), jnp.float32)

    def zero_body(i, carry):
        hist_v[pl.ds(i * 16, 16)] = zeros16
        return carry

    lax.fori_loop(0, N_PAD // 16, zero_body, 0)

    @plsc.parallel_loop(0, E_PER_TILE // 16, unroll=8)
    def count_body(k):
        idxv = dst_all[pl.ds(k * 16, 16)]
        plsc.addupdate_scatter(hist_v, [idxv], ones16)
    pltpu.sync_copy(hist_v, out_hbm.at[wid])


@functools.partial(
    pl.kernel,
    mesh=_mesh(),
    out_type=jax.ShapeDtypeStruct((N_PAD, HID_DIM), jnp.float32),
    scratch_types=[
        pltpu.VMEM((STRIP + 16,), jnp.int32),
        pltpu.VMEM((STRIP + 16,), jnp.int32),
        pltpu.VMEM((STRIP + 16,), jnp.int32),
        pltpu.VMEM((STRIP + 16,), jnp.int32),
        pltpu.VMEM((CAP,), jnp.int32),
        pltpu.VMEM((SUB,), jnp.int32),
        pltpu.VMEM((SUB,), jnp.int32),
        pltpu.VMEM((SUB, HID_DIM), jnp.float32),
        pltpu.VMEM((SUB, HID_DIM), jnp.float32),
        pltpu.VMEM((OWN + 1, HID_DIM), jnp.float32),
        pltpu.SemaphoreType.DMA,
        pltpu.SemaphoreType.DMA,
        pltpu.SemaphoreType.DMA,
        pltpu.SemaphoreType.DMA,
        pltpu.SemaphoreType.DMA,
        pltpu.SemaphoreType.DMA,
    ],
    compiler_params=pltpu.CompilerParams(needs_layout_passes=False),
)
def _msg_sc(g_hbm, src_hbm, dst_hbm, out_hbm,
            dstb0, dstb1, srcb0, srcb1, pos_l, gbuf0, gbuf1,
            rows0, rows1, acc,
            semd0, semd1, sems0, sems1, gsem0, gsem1):
    c = lax.axis_index("c")
    s = lax.axis_index("s")
    wid = c * 16 + s
    lo = wid * OWN
    iota16 = lax.iota(jnp.int32, 16)

    pltpu.sync_copy(g_hbm.at[pl.ds(lo, OWN)], acc.at[pl.ds(0, OWN)])

    dstb = (dstb0, dstb1)
    srcb = (srcb0, srcb1)
    semd = (semd0, semd1)
    sems = (sems0, sems1)
    gbuf = (gbuf0, gbuf1)
    rows = (rows0, rows1)
    gsem = (gsem0, gsem1)

    junk_dst = jnp.full((16,), lo + OWN, jnp.int32)
    junk_src = jnp.zeros((16,), jnp.int32)
    for b in range(2):
        dstb[b][pl.ds(STRIP, 16)] = junk_dst
        srcb[b][pl.ds(STRIP, 16)] = junk_src

    for b in range(2):
        pltpu.async_copy(dst_hbm.at[pl.ds(b * STRIP, STRIP)],
                         dstb[b].at[pl.ds(0, STRIP)], semd[b])
        pltpu.async_copy(src_hbm.at[pl.ds(b * STRIP, STRIP)],
                         srcb[b].at[pl.ds(0, STRIP)], sems[b])

    pad_pos = jnp.full((16,), STRIP, jnp.int32)

    def process_strip(dst_s, src_s):
        def scan_body(i, cnt):
            dvec = dst_s[pl.ds(i * 16, 16)]
            dl = dvec - lo
            m = (dl >= 0) & (dl < OWN)
            plsc.store_compressed(pos_l.at[pl.ds(cnt, 16)],
                                  i * 16 + iota16, mask=m)
            pc = plsc.all_reduce_population_count(m)
            return cnt + pc[0]

        cnt = lax.fori_loop(0, STRIP // 16, scan_body, jnp.int32(0))

        for t in range(CAP // 16 - STRIP // 16):
            pos_l[pl.ds(cnt + t * 16, 16)] = pad_pos

        n_sub = (cnt + (SUB - 1)) // SUB
        n_fr = (n_sub + 1) // 2

        def fill_and_gather(bb, sb):
            for j4 in range(SUB // 16):
                posv = pos_l[pl.ds(sb * SUB + j4 * 16, 16)]
                gbuf[bb][pl.ds(j4 * 16, 16)] = plsc.load_gather(src_s, [posv])
            pltpu.async_copy(g_hbm.at[gbuf[bb]], rows[bb], gsem[bb])

        def accumulate(bb, sb):
            def grp_body(g16, carry2):
                posv = pos_l[pl.ds(sb * SUB + g16 * 16, 16)]
                dlv = plsc.load_gather(dst_s, [posv]) - lo
                for lane in range(16):
                    dlb = jnp.take(dlv, jnp.full((16,), lane, jnp.int32))
                    e = g16 * 16 + lane

                    @plsc.parallel_loop(0, HID_DIM // 16, unroll=HID_DIM // 16)
                    def col_body(j):
                        colv = j * 16 + iota16
                        val = rows[bb][e, pl.ds(j * 16, 16)]
                        plsc.addupdate_scatter(acc, [dlb, colv], val)
                return carry2

            lax.fori_loop(0, SUB // 16, grp_body, 0)

        fill_and_gather(0, 0)

        def flush_round(fr, carry):
            for b in range(2):
                sb = 2 * fr + b
                pltpu.make_async_copy(
                    g_hbm.at[gbuf[b]], rows[b], gsem[b]).wait()
                fill_and_gather(1 - b, sb + 1)
                accumulate(b, sb)
            return carry

        lax.fori_loop(0, n_fr, flush_round, 0)
        pltpu.make_async_copy(g_hbm.at[gbuf[0]], rows[0], gsem[0]).wait()

    def round_body(r, carry):
        for b in range(2):
            i = 2 * r + b
            pltpu.make_async_copy(
                dst_hbm.at[pl.ds(i * STRIP, STRIP)],
                dstb[b].at[pl.ds(0, STRIP)], semd[b]).wait()
            pltpu.make_async_copy(
                src_hbm.at[pl.ds(i * STRIP, STRIP)],
                srcb[b].at[pl.ds(0, STRIP)], sems[b]).wait()
            process_strip(dstb[b], srcb[b])
            pltpu.async_copy(dst_hbm.at[pl.ds((i + 2) * STRIP, STRIP)],
                             dstb[b].at[pl.ds(0, STRIP)], semd[b])
            pltpu.async_copy(src_hbm.at[pl.ds((i + 2) * STRIP, STRIP)],
                             srcb[b].at[pl.ds(0, STRIP)], sems[b])
        return carry

    lax.fori_loop(0, N_STRIPS // 2, round_body, 0)
    for b in range(2):
        i = N_STRIPS + b
        pltpu.make_async_copy(
            dst_hbm.at[pl.ds(i * STRIP, STRIP)],
            dstb[b].at[pl.ds(0, STRIP)], semd[b]).wait()
        pltpu.make_async_copy(
            src_hbm.at[pl.ds(i * STRIP, STRIP)],
            srcb[b].at[pl.ds(0, STRIP)], sems[b]).wait()

    pltpu.sync_copy(acc.at[pl.ds(0, OWN)], out_hbm.at[pl.ds(lo, OWN)])


ROW_BLK = 512
GRID = N_PAD // ROW_BLK


def _dinv(deg_blk):
    return lax.rsqrt(jnp.sum(deg_blk, axis=0) + 1.0)


def _tc1_body(x_ref, w_ref, deg_ref, g_ref):
    dinv = _dinv(deg_ref[...])
    h = jnp.dot(x_ref[...], w_ref[...], preferred_element_type=jnp.float32)
    g_ref[...] = h * dinv[:, None]


def _tc1(x_p, W1, deg32):
    return pl.pallas_call(
        _tc1_body,
        grid=(GRID,),
        in_specs=[
            pl.BlockSpec((ROW_BLK, IN_DIM), lambda i: (i, 0)),
            pl.BlockSpec((IN_DIM, HID_DIM), lambda i: (0, 0)),
            pl.BlockSpec((N_TILES, ROW_BLK), lambda i: (0, i)),
        ],
        out_specs=pl.BlockSpec((ROW_BLK, HID_DIM), lambda i: (i, 0)),
        out_shape=jax.ShapeDtypeStruct((N_PAD, HID_DIM), jnp.float32),
    )(x_p, W1, deg32)


def _tc2_body(acc_ref, deg_ref, b_ref, w_ref, g_ref):
    dinv = _dinv(deg_ref[...])
    out1 = jax.nn.relu(acc_ref[...] * dinv[:, None] + b_ref[...])
    h2 = jnp.dot(out1, w_ref[...], preferred_element_type=jnp.float32)
    g_ref[...] = h2 * dinv[:, None]


def _tc2(acc1, deg32, b1r, W2):
    return pl.pallas_call(
        _tc2_body,
        grid=(GRID,),
        in_specs=[
            pl.BlockSpec((ROW_BLK, HID_DIM), lambda i: (i, 0)),
            pl.BlockSpec((N_TILES, ROW_BLK), lambda i: (0, i)),
            pl.BlockSpec((1, HID_DIM), lambda i: (0, 0)),
            pl.BlockSpec((HID_DIM, OUT_DIM), lambda i: (0, 0)),
        ],
        out_specs=pl.BlockSpec((ROW_BLK, OUT_DIM), lambda i: (i, 0)),
        out_shape=jax.ShapeDtypeStruct((N_PAD, OUT_DIM), jnp.float32),
    )(acc1, deg32, b1r, W2)


def _tc3_body(acc_ref, deg_ref, b_ref, o_ref):
    dinv = _dinv(deg_ref[...])
    o_ref[...] = acc_ref[...] * dinv[:, None] + b_ref[...]


def _tc3(acc2, deg32, b2r):
    return pl.pallas_call(
        _tc3_body,
        grid=(GRID,),
        in_specs=[
            pl.BlockSpec((ROW_BLK, OUT_DIM), lambda i: (i, 0)),
            pl.BlockSpec((N_TILES, ROW_BLK), lambda i: (0, i)),
            pl.BlockSpec((1, OUT_DIM), lambda i: (0, 0)),
        ],
        out_specs=pl.BlockSpec((ROW_BLK, OUT_DIM), lambda i: (i, 0)),
        out_shape=jax.ShapeDtypeStruct((N_PAD, OUT_DIM), jnp.float32),
    )(acc2, deg32, b2r)


def kernel(x, edge_index, W1, b1, W2, b2):
    src = edge_index[0].astype(jnp.int32)
    dst = edge_index[1].astype(jnp.int32)
    e = src.shape[0]
    src_p = jnp.pad(src, (0, E_ALLOC - e))
    dst_p = jnp.pad(dst, (0, E_ALLOC - e), constant_values=JUNK_ROW)
    x_p = jnp.pad(x, ((0, N_PAD - x.shape[0]), (0, 0)))
    b1r = b1.reshape(1, -1)
    b2r = b2.reshape(1, -1)

    deg32 = _deg_sc(dst_p)
    g1 = _tc1(x_p, W1, deg32)
    acc1 = _msg_sc(g1, src_p, dst_p)
    g2 = _tc2(acc1, deg32, b1r, W2)
    acc2 = _msg_sc(g2, src_p, dst_p)
    out = _tc3(acc2, deg32, b2r)
    return out[:N_NODES]


# speedup vs baseline: 3.3617x; 3.3617x over previous
import functools

import jax
import jax.numpy as jnp
from jax import lax
from jax.experimental import pallas as pl
from jax.experimental.pallas import tpu as pltpu
from jax.experimental.pallas import tpu_sc as plsc

N_NODES = 10000
IN_DIM = 128
HID_DIM = 256
OUT_DIM = 256

N_PAD = 10240
N_TILES = 32
OWN = N_PAD // N_TILES

STRIP = 4096
CAP = STRIP + 64
SUB = 64
E_PER_TILE = 10240
E_PAD = 327680
N_STRIPS = E_PAD // STRIP
E_ALLOC = E_PAD + 2 * STRIP
JUNK_ROW = N_NODES + 16


def _mesh():
    return plsc.VectorSubcoreMesh(core_axis_name="c", subcore_axis_name="s")


@functools.partial(
    pl.kernel,
    mesh=_mesh(),
    out_type=jax.ShapeDtypeStruct((N_TILES, N_PAD), jnp.float32),
    scratch_types=[
        pltpu.VMEM((E_PER_TILE,), jnp.int32),
        pltpu.VMEM((N_PAD,), jnp.float32),
    ],
    compiler_params=pltpu.CompilerParams(needs_layout_passes=False),
)
def _deg_sc(dst_hbm, out_hbm, dst_all, hist_v):
    c = lax.axis_index("c")
    s = lax.axis_index("s")
    wid = c * 16 + s
    e0 = wid * E_PER_TILE
    pltpu.sync_copy(dst_hbm.at[pl.ds(e0, E_PER_TILE)], dst_all)

    zeros16 = jnp.zeros((16,), jnp.float32)
    ones16 = jnp.ones((16,), jnp.float32)

    def zero_body(i, carry):
        hist_v[pl.ds(i * 16, 16)] = zeros16
        return carry

    lax.fori_loop(0, N_PAD // 16, zero_body, 0)

    def count_body(k, carry):
        idxv = dst_all[pl.ds(k * 16, 16)]
        plsc.addupdate_scatter(hist_v, [idxv], ones16)
        return carry

    lax.fori_loop(0, E_PER_TILE // 16, count_body, 0)
    pltpu.sync_copy(hist_v, out_hbm.at[wid])


@functools.partial(
    pl.kernel,
    mesh=_mesh(),
    out_type=jax.ShapeDtypeStruct((N_PAD, HID_DIM), jnp.float32),
    scratch_types=[
        pltpu.VMEM((STRIP,), jnp.int32),
        pltpu.VMEM((STRIP,), jnp.int32),
        pltpu.VMEM((STRIP,), jnp.int32),
        pltpu.VMEM((STRIP,), jnp.int32),
        pltpu.VMEM((CAP,), jnp.int32),
        pltpu.VMEM((CAP,), jnp.int32),
        pltpu.VMEM((SUB,), jnp.int32),
        pltpu.VMEM((SUB, HID_DIM), jnp.float32),
        pltpu.VMEM((OWN + 1, HID_DIM), jnp.float32),
        pltpu.SemaphoreType.DMA,
        pltpu.SemaphoreType.DMA,
        pltpu.SemaphoreType.DMA,
        pltpu.SemaphoreType.DMA,
        pltpu.SemaphoreType.DMA,
    ],
    compiler_params=pltpu.CompilerParams(needs_layout_passes=False),
)
def _msg_sc(g_hbm, src_hbm, dst_hbm, out_hbm,
            dstb0, dstb1, srcb0, srcb1, dl_l, src_l, gbuf, rows, acc,
            semd0, semd1, sems0, sems1, gsem):
    c = lax.axis_index("c")
    s = lax.axis_index("s")
    wid = c * 16 + s
    lo = wid * OWN
    iota16 = lax.iota(jnp.int32, 16)

    pltpu.sync_copy(g_hbm.at[pl.ds(lo, OWN)], acc.at[pl.ds(0, OWN)])

    dstb = (dstb0, dstb1)
    srcb = (srcb0, srcb1)
    semd = (semd0, semd1)
    sems = (sems0, sems1)

    for b in range(2):
        pltpu.async_copy(dst_hbm.at[pl.ds(b * STRIP, STRIP)], dstb[b], semd[b])
        pltpu.async_copy(src_hbm.at[pl.ds(b * STRIP, STRIP)], srcb[b], sems[b])

    dummy_dl = jnp.full((16,), OWN, jnp.int32)
    dummy_src = jnp.zeros((16,), jnp.int32)

    def process_strip(dst_s, src_s):
        def scan_body(i, cnt):
            dvec = dst_s[pl.ds(i * 16, 16)]
            svec = src_s[pl.ds(i * 16, 16)]
            dl = dvec - lo
            m = (dl >= 0) & (dl < OWN)
            plsc.store_compressed(dl_l.at[pl.ds(cnt, 16)], dl, mask=m)
            plsc.store_compressed(src_l.at[pl.ds(cnt, 16)], svec, mask=m)
            pc = plsc.all_reduce_population_count(m)
            return cnt + pc[0]

        cnt = lax.fori_loop(0, STRIP // 16, scan_body, jnp.int32(0))

        for t in range(SUB // 16):
            dl_l[pl.ds(cnt + t * 16, 16)] = dummy_dl
            src_l[pl.ds(cnt + t * 16, 16)] = dummy_src

        n_sub = (cnt + (SUB - 1)) // SUB

        def flush_body(sb, carry):
            for j4 in range(SUB // 16):
                gbuf[pl.ds(j4 * 16, 16)] = src_l[pl.ds(sb * SUB + j4 * 16, 16)]
            pltpu.async_copy(g_hbm.at[gbuf], rows, gsem).wait()

            def grp_body(g16, carry2):
                dlvec = dl_l[pl.ds(sb * SUB + g16 * 16, 16)]
                for lane in range(16):
                    dlb = jnp.take(dlvec, jnp.full((16,), lane, jnp.int32))
                    e = g16 * 16 + lane
                    for j in range(HID_DIM // 16):
                        colv = (j * 16) + iota16
                        val = rows[e, pl.ds(j * 16, 16)]
                        plsc.addupdate_scatter(acc, [dlb, colv], val)
                return carry2

            lax.fori_loop(0, SUB // 16, grp_body, 0)
            return carry

        lax.fori_loop(0, n_sub, flush_body, 0)

    def round_body(r, carry):
        for b in range(2):
            i = 2 * r + b
            pltpu.make_async_copy(
                dst_hbm.at[pl.ds(i * STRIP, STRIP)], dstb[b], semd[b]).wait()
            pltpu.make_async_copy(
                src_hbm.at[pl.ds(i * STRIP, STRIP)], srcb[b], sems[b]).wait()
            process_strip(dstb[b], srcb[b])
            pltpu.async_copy(
                dst_hbm.at[pl.ds((i + 2) * STRIP, STRIP)], dstb[b], semd[b])
            pltpu.async_copy(
                src_hbm.at[pl.ds((i + 2) * STRIP, STRIP)], srcb[b], sems[b])
        return carry

    lax.fori_loop(0, N_STRIPS // 2, round_body, 0)
    for b in range(2):
        i = N_STRIPS + b
        pltpu.make_async_copy(
            dst_hbm.at[pl.ds(i * STRIP, STRIP)], dstb[b], semd[b]).wait()
        pltpu.make_async_copy(
            src_hbm.at[pl.ds(i * STRIP, STRIP)], srcb[b], sems[b]).wait()

    pltpu.sync_copy(acc.at[pl.ds(0, OWN)], out_hbm.at[pl.ds(lo, OWN)])


ROW_BLK = 512
GRID = N_PAD // ROW_BLK


def _dinv(deg_blk):
    return lax.rsqrt(jnp.sum(deg_blk, axis=0) + 1.0)


def _tc1_body(x_ref, w_ref, deg_ref, g_ref):
    dinv = _dinv(deg_ref[...])
    h = jnp.dot(x_ref[...], w_ref[...], preferred_element_type=jnp.float32)
    g_ref[...] = h * dinv[:, None]


def _tc1(x_p, W1, deg32):
    return pl.pallas_call(
        _tc1_body,
        grid=(GRID,),
        in_specs=[
            pl.BlockSpec((ROW_BLK, IN_DIM), lambda i: (i, 0)),
            pl.BlockSpec((IN_DIM, HID_DIM), lambda i: (0, 0)),
            pl.BlockSpec((N_TILES, ROW_BLK), lambda i: (0, i)),
        ],
        out_specs=pl.BlockSpec((ROW_BLK, HID_DIM), lambda i: (i, 0)),
        out_shape=jax.ShapeDtypeStruct((N_PAD, HID_DIM), jnp.float32),
    )(x_p, W1, deg32)


def _tc2_body(acc_ref, deg_ref, b_ref, w_ref, g_ref):
    dinv = _dinv(deg_ref[...])
    out1 = jax.nn.relu(acc_ref[...] * dinv[:, None] + b_ref[...])
    h2 = jnp.dot(out1, w_ref[...], preferred_element_type=jnp.float32)
    g_ref[...] = h2 * dinv[:, None]


def _tc2(acc1, deg32, b1r, W2):
    return pl.pallas_call(
        _tc2_body,
        grid=(GRID,),
        in_specs=[
            pl.BlockSpec((ROW_BLK, HID_DIM), lambda i: (i, 0)),
            pl.BlockSpec((N_TILES, ROW_BLK), lambda i: (0, i)),
            pl.BlockSpec((1, HID_DIM), lambda i: (0, 0)),
            pl.BlockSpec((HID_DIM, OUT_DIM), lambda i: (0, 0)),
        ],
        out_specs=pl.BlockSpec((ROW_BLK, OUT_DIM), lambda i: (i, 0)),
        out_shape=jax.ShapeDtypeStruct((N_PAD, OUT_DIM), jnp.float32),
    )(acc1, deg32, b1r, W2)


def _tc3_body(acc_ref, deg_ref, b_ref, o_ref):
    dinv = _dinv(deg_ref[...])
    o_ref[...] = acc_ref[...] * dinv[:, None] + b_ref[...]


def _tc3(acc2, deg32, b2r):
    return pl.pallas_call(
        _tc3_body,
        grid=(GRID,),
        in_specs=[
            pl.BlockSpec((ROW_BLK, OUT_DIM), lambda i: (i, 0)),
            pl.BlockSpec((N_TILES, ROW_BLK), lambda i: (0, i)),
            pl.BlockSpec((1, OUT_DIM), lambda i: (0, 0)),
        ],
        out_specs=pl.BlockSpec((ROW_BLK, OUT_DIM), lambda i: (i, 0)),
        out_shape=jax.ShapeDtypeStruct((N_PAD, OUT_DIM), jnp.float32),
    )(acc2, deg32, b2r)


def kernel(x, edge_index, W1, b1, W2, b2):
    src = edge_index[0].astype(jnp.int32)
    dst = edge_index[1].astype(jnp.int32)
    e = src.shape[0]
    src_p = jnp.pad(src, (0, E_ALLOC - e))
    dst_p = jnp.pad(dst, (0, E_ALLOC - e), constant_values=JUNK_ROW)
    x_p = jnp.pad(x, ((0, N_PAD - x.shape[0]), (0, 0)))
    b1r = b1.reshape(1, -1)
    b2r = b2.reshape(1, -1)

    deg32 = _deg_sc(dst_p)
    g1 = _tc1(x_p, W1, deg32)
    acc1 = _msg_sc(g1, src_p, dst_p)
    g2 = _tc2(acc1, deg32, b1r, W2)
    acc2 = _msg_sc(g2, src_p, dst_p)
    out = _tc3(acc2, deg32, b2r)
    return out[:N_NODES]
